# Initial kernel scaffold; baseline (speedup 1.0000x reference)
import jax
import jax.numpy as jnp
from jax.experimental import pallas as pl


def kernel(x, adj, W0, W1):
    raise NotImplementedError("write your pallas kernel here")


# speedup vs baseline: 1.4367x; 1.4367x over previous
import functools

import jax
import jax.numpy as jnp
from jax import lax
from jax.experimental import pallas as pl
from jax.experimental.pallas import tpu as pltpu
from jax.experimental.pallas import tpu_sc as plsc

_N, _K, _D = 10000, 32, 128
_NW = 32
_C = 8
_G = 128
_NPAD = 10240
_NODES_PER_W = _NPAD // _NW
_CHUNKS = _NODES_PER_W // _C
_ROWS = _C * _K
_GATHERS = _ROWS // _G
_LANES = 16
_DCH = _D // _LANES


def _mean_body(table, idx, out, idx_v, rows_v, out_v, sem):
    cid = lax.axis_index("c")
    sid = lax.axis_index("s")
    wid = sid * 2 + cid

    pltpu.sync_copy(idx.at[wid], idx_v)

    def chunk_body(it, _):
        for p in range(_GATHERS):
            pltpu.async_copy(
                table.at[idx_v.at[it * _GATHERS + p]],
                rows_v.at[pl.ds(p * _G, _G)],
                sem)
        for p in range(_GATHERS):
            pltpu.make_async_copy(
                table.at[idx_v.at[it * _GATHERS + p]],
                rows_v.at[pl.ds(p * _G, _G)],
                sem).wait()

        def node_body(c, _):
            base = c * _K

            def k_body(k, accs):
                r = base + k
                return tuple(accs[j] + rows_v[r, pl.ds(j * _LANES, _LANES)]
                             for j in range(_DCH))

            accs = lax.fori_loop(
                0, _K, k_body,
                tuple(jnp.zeros((_LANES,), jnp.float32) for _ in range(_DCH)))
            for j in range(_DCH):
                out_v[c, pl.ds(j * _LANES, _LANES)] = accs[j] * (1.0 / _K)
            return 0

        lax.fori_loop(0, _C, node_body, 0)
        pltpu.sync_copy(
            out_v, out.at[pl.ds(wid * _NODES_PER_W + it * _C, _C)])
        return 0

    lax.fori_loop(0, _CHUNKS, chunk_body, 0)


def _sc_mean(table, idx3):
    mesh = plsc.VectorSubcoreMesh(core_axis_name="c", subcore_axis_name="s")
    return pl.kernel(
        _mean_body,
        out_type=jax.ShapeDtypeStruct((_NPAD, _D), jnp.float32),
        mesh=mesh,
        scratch_types=[
            pltpu.VMEM((_CHUNKS * _GATHERS, _G), jnp.int32),
            pltpu.VMEM((_ROWS, _D), jnp.float32),
            pltpu.VMEM((_C, _D), jnp.float32),
            pltpu.SemaphoreType.DMA,
        ],
    )(table, idx3)


_BLK = 1024


def _linear_body(m_ref, wt_ref, o_ref):
    h = jnp.dot(m_ref[...], wt_ref[...], preferred_element_type=jnp.float32)
    h = jnp.maximum(h, 0.0)
    n = jnp.sqrt(jnp.sum(h * h, axis=-1, keepdims=True))
    o_ref[...] = h / jnp.maximum(n, 1e-12)


def _tc_linear(m, wt):
    return pl.pallas_call(
        _linear_body,
        grid=(_NPAD // _BLK,),
        in_specs=[
            pl.BlockSpec((_BLK, _D), lambda i: (i, 0)),
            pl.BlockSpec((_D, _D), lambda i: (0, 0)),
        ],
        out_specs=pl.BlockSpec((_BLK, _D), lambda i: (i, 0)),
        out_shape=jax.ShapeDtypeStruct((_NPAD, _D), jnp.float32),
    )(m, wt)


def kernel(x, adj, W0, W1):
    adj_p = jnp.concatenate(
        [adj, jnp.zeros((_NPAD - _N, _K), jnp.int32)], axis=0)
    idx3 = adj_p.reshape(_NW, _CHUNKS * _GATHERS, _G)

    m1 = _sc_mean(x, idx3)
    h1 = _tc_linear(m1, W0.T)
    m2 = _sc_mean(h1, idx3)
    h2 = _tc_linear(m2, W1.T)
    return h2[:_N]


# speedup vs baseline: 1.5898x; 1.1066x over previous
import functools

import jax
import jax.numpy as jnp
from jax import lax
from jax.experimental import pallas as pl
from jax.experimental.pallas import tpu as pltpu
from jax.experimental.pallas import tpu_sc as plsc

_N, _K, _D = 10000, 32, 128
_NW = 32
_C = 8
_G = 128
_NPAD = 10240
_NODES_PER_W = _NPAD // _NW
_CHUNKS = _NODES_PER_W // _C
_ROWS = _C * _K
_GATHERS = _ROWS // _G
_LANES = 16
_DCH = _D // _LANES


def _mean_body(table, idx, out, idx_v, rows_v, out_v, sem0, sem1):
    cid = lax.axis_index("c")
    sid = lax.axis_index("s")
    wid = sid * 2 + cid

    pltpu.sync_copy(idx.at[wid], idx_v)

    sems = (sem0, sem1)

    def start(it, par):
        for p in range(_GATHERS):
            pltpu.async_copy(
                table.at[idx_v.at[it * _GATHERS + p]],
                rows_v.at[pl.ds(par * _ROWS + p * _G, _G)],
                sems[par])

    def wait(it, par):
        for p in range(_GATHERS):
            pltpu.make_async_copy(
                table.at[idx_v.at[it * _GATHERS + p]],
                rows_v.at[pl.ds(par * _ROWS + p * _G, _G)],
                sems[par]).wait()

    def compute(it, par):
        def node_body(c, _):
            base = par * _ROWS + c * _K
            accs = [rows_v[base, pl.ds(j * _LANES, _LANES)]
                    for j in range(_DCH)]
            for k in range(1, _K):
                for j in range(_DCH):
                    accs[j] = accs[j] + rows_v[base + k,
                                               pl.ds(j * _LANES, _LANES)]
            for j in range(_DCH):
                out_v[c, pl.ds(j * _LANES, _LANES)] = accs[j] * (1.0 / _K)
            return 0

        lax.fori_loop(0, _C, node_body, 0)
        pltpu.sync_copy(
            out_v, out.at[pl.ds(wid * _NODES_PER_W + it * _C, _C)])

    start(0, 0)

    def pair_body(t, _):
        it = 2 * t
        start(it + 1, 1)
        wait(it, 0)
        compute(it, 0)

        @pl.when(t + 1 < _CHUNKS // 2)
        def _():
            start(it + 2, 0)

        wait(it + 1, 1)
        compute(it + 1, 1)
        return 0

    lax.fori_loop(0, _CHUNKS // 2, pair_body, 0)


def _sc_mean(table, idx3):
    mesh = plsc.VectorSubcoreMesh(core_axis_name="c", subcore_axis_name="s")
    return pl.kernel(
        _mean_body,
        out_type=jax.ShapeDtypeStruct((_NPAD, _D), jnp.float32),
        mesh=mesh,
        scratch_types=[
            pltpu.VMEM((_CHUNKS * _GATHERS, _G), jnp.int32),
            pltpu.VMEM((2 * _ROWS, _D), jnp.float32),
            pltpu.VMEM((_C, _D), jnp.float32),
            pltpu.SemaphoreType.DMA,
            pltpu.SemaphoreType.DMA,
        ],
    )(table, idx3)


_BLK = 1024


def _linear_body(m_ref, wt_ref, o_ref):
    h = jnp.dot(m_ref[...], wt_ref[...], preferred_element_type=jnp.float32)
    h = jnp.maximum(h, 0.0)
    n = jnp.sqrt(jnp.sum(h * h, axis=-1, keepdims=True))
    o_ref[...] = h / jnp.maximum(n, 1e-12)


def _tc_linear(m, wt):
    return pl.pallas_call(
        _linear_body,
        grid=(_NPAD // _BLK,),
        in_specs=[
            pl.BlockSpec((_BLK, _D), lambda i: (i, 0)),
            pl.BlockSpec((_D, _D), lambda i: (0, 0)),
        ],
        out_specs=pl.BlockSpec((_BLK, _D), lambda i: (i, 0)),
        out_shape=jax.ShapeDtypeStruct((_NPAD, _D), jnp.float32),
    )(m, wt)


def kernel(x, adj, W0, W1):
    adj_p = jnp.concatenate(
        [adj, jnp.zeros((_NPAD - _N, _K), jnp.int32)], axis=0)
    idx3 = adj_p.reshape(_NW, _CHUNKS * _GATHERS, _G)

    m1 = _sc_mean(x, idx3)
    h1 = _tc_linear(m1, W0.T)
    m2 = _sc_mean(h1, idx3)
    h2 = _tc_linear(m2, W1.T)
    return h2[:_N]


# speedup vs baseline: 4.1768x; 2.6273x over previous
import jax
import jax.numpy as jnp
from jax import lax
from jax.experimental import pallas as pl
from jax.experimental.pallas import tpu as pltpu
from jax.experimental.pallas import tpu_sc as plsc

_N, _K, _D = 10000, 32, 128
_NT = 32
_FPT = _D // _NT
_NPAD = 10240
_BN = 512
_NB = _NPAD // _BN
_LANES = 16
_GRP = _BN // _LANES


def _mean_body(tbl, adjb, out, tbl_v, adj_v, out_v, sem_t, sem_a, sem_o):
    cid = lax.axis_index("c")
    sid = lax.axis_index("s")
    wid = sid * 2 + cid

    ct = pltpu.async_copy(tbl.at[wid], tbl_v, sem_t)
    pltpu.async_copy(adjb.at[0], adj_v.at[0], sem_a)
    ct.wait()


    def block_body(b, _):
        par = jnp.bitwise_and(b, 1)
        pltpu.make_async_copy(adjb.at[b], adj_v.at[par], sem_a).wait()

        @pl.when(b + 1 < _NB)
        def _():
            pltpu.async_copy(adjb.at[b + 1], adj_v.at[1 - par], sem_a)

        @pl.when(b >= 2)
        def _():
            pltpu.make_async_copy(
                out_v.at[par], out.at[wid, b - 2], sem_o).wait()

        def group_body(g, _):
            accs = None
            for k in range(_K):
                idxv = adj_v[par, k, pl.ds(g * _LANES, _LANES)] * _FPT
                vals = [plsc.load_gather(tbl_v, [idxv + f])
                        for f in range(_FPT)]
                if accs is None:
                    accs = vals
                else:
                    accs = [a + v for a, v in zip(accs, vals)]
            for f in range(_FPT):
                out_v[par, f, pl.ds(g * _LANES, _LANES)] = (
                    accs[f] * (1.0 / _K))
            return 0

        lax.fori_loop(0, _GRP, group_body, 0)
        pltpu.async_copy(out_v.at[par], out.at[wid, b], sem_o)
        return 0

    lax.fori_loop(0, _NB, block_body, 0)
    pltpu.make_async_copy(out_v.at[0], out.at[wid, _NB - 2], sem_o).wait()
    pltpu.make_async_copy(out_v.at[1], out.at[wid, _NB - 1], sem_o).wait()


def _sc_mean(tbl_slab, adjb):
    mesh = plsc.VectorSubcoreMesh(core_axis_name="c", subcore_axis_name="s")
    return pl.kernel(
        _mean_body,
        out_type=jax.ShapeDtypeStruct((_NT, _NB, _FPT, _BN), jnp.float32),
        mesh=mesh,
        compiler_params=pltpu.CompilerParams(needs_layout_passes=False),
        scratch_types=[
            pltpu.VMEM((_NPAD * _FPT,), jnp.float32),
            pltpu.VMEM((2, _K, _BN), jnp.int32),
            pltpu.VMEM((2, _FPT, _BN), jnp.float32),
            pltpu.SemaphoreType.DMA,
            pltpu.SemaphoreType.DMA,
            pltpu.SemaphoreType.DMA,
        ],
    )(tbl_slab, adjb)


_BLK = 1024


def _linear_body(m_ref, wt_ref, o_ref):
    h = jnp.dot(m_ref[...], wt_ref[...], preferred_element_type=jnp.float32)
    h = jnp.maximum(h, 0.0)
    n = jnp.sqrt(jnp.sum(h * h, axis=-1, keepdims=True))
    o_ref[...] = h / jnp.maximum(n, 1e-12)


def _tc_linear(m, wt):
    return pl.pallas_call(
        _linear_body,
        grid=(_NPAD // _BLK,),
        in_specs=[
            pl.BlockSpec((_BLK, _D), lambda i: (i, 0)),
            pl.BlockSpec((_D, _D), lambda i: (0, 0)),
        ],
        out_specs=pl.BlockSpec((_BLK, _D), lambda i: (i, 0)),
        out_shape=jax.ShapeDtypeStruct((_NPAD, _D), jnp.float32),
    )(m, wt)


def _to_slab(h):
    return h.reshape(_NPAD, _NT, _FPT).transpose(1, 0, 2).reshape(
        _NT, _NPAD * _FPT)


def _from_slab(o):
    return o.transpose(1, 3, 0, 2).reshape(_NPAD, _D)


def kernel(x, adj, W0, W1):
    adj_p = jnp.concatenate(
        [adj, jnp.zeros((_NPAD - _N, _K), jnp.int32)], axis=0)
    adjb = adj_p.reshape(_NB, _BN, _K).transpose(0, 2, 1)
    x_p = jnp.concatenate(
        [x, jnp.zeros((_NPAD - _N, _D), jnp.float32)], axis=0)

    m1 = _from_slab(_sc_mean(_to_slab(x_p), adjb))
    h1 = _tc_linear(m1, W0.T)
    m2 = _from_slab(_sc_mean(_to_slab(h1), adjb))
    h2 = _tc_linear(m2, W1.T)
    return h2[:_N]


# speedup vs baseline: 5.6196x; 1.3454x over previous
import jax
import jax.numpy as jnp
from jax import lax
from jax.experimental import pallas as pl
from jax.experimental.pallas import tpu as pltpu
from jax.experimental.pallas import tpu_sc as plsc

_N, _K, _D = 10000, 32, 128
_NT = 32
_FPT = _D // _NT
_NPAD = 10240
_BN = 512
_NB = _NPAD // _BN
_LANES = 16
_GRP = _BN // _LANES


def _mean_body(tbl, adjb, out, tbl_v, adj_v, out_v, sem_t, sem_a, sem_o):
    cid = lax.axis_index("c")
    sid = lax.axis_index("s")
    wid = sid * 2 + cid

    ct = pltpu.async_copy(tbl.at[wid], tbl_v, sem_t)
    pltpu.async_copy(adjb.at[0], adj_v.at[0], sem_a)
    ct.wait()

    def block_body(b, _):
        par = jnp.bitwise_and(b, 1)
        pltpu.make_async_copy(adjb.at[b], adj_v.at[par], sem_a).wait()

        @pl.when(b + 1 < _NB)
        def _():
            pltpu.async_copy(adjb.at[b + 1], adj_v.at[1 - par], sem_a)

        @pl.when(b >= 2)
        def _():
            pltpu.make_async_copy(
                out_v.at[par], out.at[wid, b - 2], sem_o).wait()

        def group_body(g, _):
            accs = None
            for k in range(_K):
                idxv = adj_v[par, k, pl.ds(g * _LANES, _LANES)]
                vals = [plsc.load_gather(tbl_v, [idxv + (f * _NPAD)])
                        for f in range(_FPT)]
                if accs is None:
                    accs = vals
                else:
                    accs = [a + v for a, v in zip(accs, vals)]
            for f in range(_FPT):
                out_v[par, f, pl.ds(g * _LANES, _LANES)] = (
                    accs[f] * (1.0 / _K))
            return 0

        lax.fori_loop(0, _GRP, group_body, 0)
        pltpu.async_copy(out_v.at[par], out.at[wid, b], sem_o)
        return 0

    lax.fori_loop(0, _NB, block_body, 0)
    pltpu.make_async_copy(out_v.at[0], out.at[wid, _NB - 2], sem_o).wait()
    pltpu.make_async_copy(out_v.at[1], out.at[wid, _NB - 1], sem_o).wait()


def _sc_mean(tbl_slab, adjb):
    mesh = plsc.VectorSubcoreMesh(core_axis_name="c", subcore_axis_name="s")
    return pl.kernel(
        _mean_body,
        out_type=jax.ShapeDtypeStruct((_NT, _NB, _FPT, _BN), jnp.float32),
        mesh=mesh,
        compiler_params=pltpu.CompilerParams(needs_layout_passes=False),
        scratch_types=[
            pltpu.VMEM((_FPT * _NPAD,), jnp.float32),
            pltpu.VMEM((2, _K, _BN), jnp.int32),
            pltpu.VMEM((2, _FPT, _BN), jnp.float32),
            pltpu.SemaphoreType.DMA,
            pltpu.SemaphoreType.DMA,
            pltpu.SemaphoreType.DMA,
        ],
    )(tbl_slab, adjb)


def _linear_body(m_ref, w_ref, o_ref):
    mt = m_ref[...].reshape(_D, _BN)
    ht = jnp.dot(w_ref[...], mt, preferred_element_type=jnp.float32)
    ht = jnp.maximum(ht, 0.0)
    nrm = jnp.sqrt(jnp.sum(ht * ht, axis=0, keepdims=True))
    o_ref[...] = ht / jnp.maximum(nrm, 1e-12)


def _tc_linear(m, w):
    return pl.pallas_call(
        _linear_body,
        grid=(_NB,),
        in_specs=[
            pl.BlockSpec((_NT, 1, _FPT, _BN), lambda b: (0, b, 0, 0)),
            pl.BlockSpec((_D, _D), lambda b: (0, 0)),
        ],
        out_specs=pl.BlockSpec((_D, _BN), lambda b: (0, b)),
        out_shape=jax.ShapeDtypeStruct((_D, _NPAD), jnp.float32),
    )(m, w)


def kernel(x, adj, W0, W1):
    adj_p = jnp.concatenate(
        [adj, jnp.zeros((_NPAD - _N, _K), jnp.int32)], axis=0)
    adjb = adj_p.reshape(_NB, _BN, _K).transpose(0, 2, 1)
    x_p = jnp.concatenate(
        [x, jnp.zeros((_NPAD - _N, _D), jnp.float32)], axis=0)
    x_slab = x_p.T.reshape(_NT, _FPT * _NPAD)

    m1 = _sc_mean(x_slab, adjb)
    h1 = _tc_linear(m1, W0)
    m2 = _sc_mean(h1.reshape(_NT, _FPT * _NPAD), adjb)
    h2 = _tc_linear(m2, W1)
    return h2.T[:_N]
